# baseline (speedup 1.0000x reference)
import functools

import jax
import jax.numpy as jnp
from jax import lax
from jax.experimental import pallas as pl
from jax.experimental.pallas import tpu as pltpu
from jax.experimental.pallas import tpu_sc as plsc

N = 10000
E = 320000
D = 128
DH = 64
NPAD = 10240
BLK = 128
NBLK = 2560
EPAD = NBLK * BLK
BPT_FULL = NBLK // 16
BPT_HALF = NBLK // 32
ROWS_PT = NPAD // 16

BN = 1000

_f32 = jnp.float32
_i32 = jnp.int32


@functools.cache
def _mesh():
    return plsc.VectorSubcoreMesh(core_axis_name="c", subcore_axis_name="s")


@functools.cache
def _sc_params():
    return pltpu.CompilerParams(needs_layout_passes=False, use_tc_tiling_on_sc=False)


def _zero_vmem2d(buf, nrows, ncols):
    z = jnp.zeros((16,), _f32)

    def zr(i, _):
        for f in range(ncols // 16):
            buf[i, pl.ds(f * 16, 16)] = z
        return 0

    lax.fori_loop(0, nrows, zr, 0)


CHK = 16


def _scale_block(buf, normv2, cb, jl):
    def grp(g, _):
        for u in range(8):
            e = g * 8 + u
            nrm = plsc.load_gather(
                normv2, [jnp.full((16,), cb, _i32), jnp.full((16,), jl, _i32),
                         jnp.full((16,), e, _i32)])
            for f in range(DH // 16):
                sl = pl.ds(f * 16, 16)
                buf[e, sl] = buf[e, sl] * nrm
        return 0

    lax.fori_loop(0, BLK // 8, grp, 0)


def _half_pass(v_hbm, out_hbm, rowb, colb, normb, base, rowv2, colv2,
               normv2, rbufs, tb, acc, gsems, ssems, isem, s, nblocks):
    nchunks = nblocks // CHK
    _zero_vmem2d(rbufs[3], BLK, DH)
    for k in range(ROWS_PT // BLK):
        pltpu.sync_copy(rbufs[3], acc.at[pl.ds(s * ROWS_PT + k * BLK, BLK)])
    @pl.when(s < 15)
    def _():
        pltpu.sync_copy(v_hbm.at[pl.ds(s * ROWS_PT, ROWS_PT)],
                        tb.at[pl.ds(s * ROWS_PT, ROWS_PT)])

    @pl.when(s == 15)
    def _():
        pltpu.sync_copy(v_hbm.at[pl.ds(15 * ROWS_PT, N - 15 * ROWS_PT)],
                        tb.at[pl.ds(15 * ROWS_PT, N - 15 * ROWS_PT)])

    plsc.subcore_barrier()

    pltpu.sync_copy(rowb.at[pl.ds(base, CHK)], rowv2.at[0])
    pltpu.sync_copy(colb.at[pl.ds(base, CHK)], colv2.at[0])
    pltpu.sync_copy(normb.at[pl.ds(base, CHK)], normv2.at[0])
    pltpu.async_copy(tb.at[colv2.at[0, 0]], rbufs[0], gsems[0])
    pltpu.async_copy(tb.at[colv2.at[0, 1]], rbufs[1], gsems[1])

    def quad(p, _):
        i1 = (p >> 2) + 1
        cb1 = i1 & 1
        off = base + i1 * CHK

        @pl.when(((p & 3) == 0) & (i1 < nchunks))
        def _():
            pltpu.async_copy(rowb.at[pl.ds(off, CHK)], rowv2.at[cb1], isem)
            pltpu.async_copy(colb.at[pl.ds(off, CHK)], colv2.at[cb1], isem)
            pltpu.async_copy(normb.at[pl.ds(off, CHK)], normv2.at[cb1], isem)

        @pl.when(((p & 3) == 3) & (i1 < nchunks))
        def _():
            pltpu.make_async_copy(rowb.at[pl.ds(off, CHK)], rowv2.at[cb1],
                                  isem).wait()
            pltpu.make_async_copy(colb.at[pl.ds(off, CHK)], colv2.at[cb1],
                                  isem).wait()
            pltpu.make_async_copy(normb.at[pl.ds(off, CHK)], normv2.at[cb1],
                                  isem).wait()

        for u in range(4):
            q = 4 * p + u
            cb = (q >> 4) & 1
            jl = q & (CHK - 1)
            w = (u + 2) % 4
            pltpu.make_async_copy(tb.at[colv2.at[cb, jl]], rbufs[u],
                                  gsems[u]).wait()
            _scale_block(rbufs[u], normv2, cb, jl)
            pltpu.async_copy(rbufs[u], acc.at[rowv2.at[cb, jl]], ssems[u],
                             add=True)

            @pl.when(q >= 2)
            def _():
                pltpu.make_async_copy(rbufs[w], acc.at[pl.ds(0, BLK)],
                                      ssems[w]).wait()

            @pl.when(q + 2 < nblocks)
            def _():
                q2 = q + 2
                pltpu.async_copy(tb.at[colv2.at[(q2 >> 4) & 1,
                                                q2 & (CHK - 1)]],
                                 rbufs[w], gsems[w])
        return 0

    lax.fori_loop(0, nblocks // 4, quad, 0)
    pltpu.make_async_copy(rbufs[(nblocks - 2) % 4], acc.at[pl.ds(0, BLK)],
                          ssems[(nblocks - 2) % 4]).wait()
    pltpu.make_async_copy(rbufs[(nblocks - 1) % 4], acc.at[pl.ds(0, BLK)],
                          ssems[(nblocks - 1) % 4]).wait()
    plsc.subcore_barrier()
    pltpu.sync_copy(acc.at[pl.ds(s * ROWS_PT, ROWS_PT)],
                    out_hbm.at[pl.ds(s * ROWS_PT, ROWS_PT)])
    plsc.subcore_barrier()


def _deg_body(rowb, wb, out, rowv, wv, zbuf, acc, sem):
    c = lax.axis_index("c")
    s = lax.axis_index("s")
    z = jnp.zeros((16,), _f32)

    def zr(i, _):
        zbuf[pl.ds(i * 16, 16)] = z
        return 0

    lax.fori_loop(0, ROWS_PT // 16, zr, 0)
    pltpu.sync_copy(zbuf, acc.at[pl.ds(s * ROWS_PT, ROWS_PT)])
    plsc.subcore_barrier()
    base = (c * 16 + s) * BPT_HALF
    pltpu.sync_copy(rowb.at[pl.ds(base, BPT_HALF)], rowv)
    pltpu.sync_copy(wb.at[pl.ds(base, BPT_HALF)], wv)

    def block(j, _):
        pltpu.sync_copy(wv.at[j], acc.at[rowv.at[j]], add=True)
        return 0

    lax.fori_loop(0, BPT_HALF, block, 0)
    plsc.subcore_barrier()
    pltpu.sync_copy(acc.at[pl.ds(s * ROWS_PT, ROWS_PT)],
                    out.at[c, pl.ds(s * ROWS_PT, ROWS_PT)])


@functools.cache
def _deg_call():
    return pl.kernel(
        _deg_body, mesh=_mesh(), compiler_params=_sc_params(),
        out_type=jax.ShapeDtypeStruct((2, NPAD), _f32),
        scratch_types=[
            pltpu.VMEM((BPT_HALF, BLK), _i32),
            pltpu.VMEM((BPT_HALF, BLK), _f32),
            pltpu.VMEM((ROWS_PT,), _f32),
            pltpu.VMEM_SHARED((NPAD,), _f32),
            pltpu.SemaphoreType.DMA,
        ],
    )


def _norm_body(rowb, colb, wb, degp, out, rowv, colv, wv, dgv, dv, nb, sem):
    c = lax.axis_index("c")
    s = lax.axis_index("s")
    base = (c * 16 + s) * BPT_HALF
    pltpu.sync_copy(rowb.at[pl.ds(base, BPT_HALF)], rowv)
    pltpu.sync_copy(colb.at[pl.ds(base, BPT_HALF)], colv)
    pltpu.sync_copy(wb.at[pl.ds(base, BPT_HALF)], wv)
    pltpu.sync_copy(degp.at[0], dgv)
    pltpu.sync_copy(degp.at[1], dv)

    def dinv_grp(i, _):
        r = i >> 3
        g = i & 7
        sl = pl.ds(g * 16, 16)
        d = dgv[r, sl] + dv[r, sl]
        y = plsc.bitcast(
            jnp.int32(0x5F3759DF) - (plsc.bitcast(d, _i32) >> 1), _f32)
        h = -0.5 * d
        y = y * (1.5 + h * y * y)
        y = y * (1.5 + h * y * y)
        y = y * (1.5 + h * y * y)
        dv[r, sl] = jnp.where(d > 0, y, 0.0)
        return 0

    lax.fori_loop(0, (NPAD // BLK) * 8, dinv_grp, 0)

    def block(j, _):
        for g in range(8):
            sl = pl.ds(g * 16, 16)
            r = rowv[j, sl]
            cc = colv[j, sl]
            dr = plsc.load_gather(dv, [r >> 7, r & 127])
            dc = plsc.load_gather(dv, [cc >> 7, cc & 127])
            nb[j, sl] = dr * wv[j, sl] * dc
        return 0

    lax.fori_loop(0, BPT_HALF, block, 0)
    pltpu.sync_copy(nb, out.at[pl.ds(base, BPT_HALF)])


@functools.cache
def _norm_call():
    return pl.kernel(
        _norm_body, mesh=_mesh(), compiler_params=_sc_params(),
        out_type=jax.ShapeDtypeStruct((NBLK, BLK), _f32),
        scratch_types=[
            pltpu.VMEM((BPT_HALF, BLK), _i32),
            pltpu.VMEM((BPT_HALF, BLK), _i32),
            pltpu.VMEM((BPT_HALF, BLK), _f32),
            pltpu.VMEM((NPAD // BLK, BLK), _f32),
            pltpu.VMEM((NPAD // BLK, BLK), _f32),
            pltpu.VMEM((BPT_HALF, BLK), _f32),
            pltpu.SemaphoreType.DMA,
        ],
    )


def _s2_body(vx0, vx1, vh0, vh1, rowb, colb, normb,
             outx10, outx11, outx20, outx21, outh10, outh11, outh20, outh21,
             rowv2, colv2, normv2, rb0, rb1, rb2, rb3, tb, acc,
             g0, g1, g2, g3, s0, s1, s2, s3, isem):
    c = lax.axis_index("c")
    s = lax.axis_index("s")

    args = (rowb, colb, normb, s * BPT_FULL, rowv2, colv2, normv2,
            (rb0, rb1, rb2, rb3), tb, acc,
            (g0, g1, g2, g3), (s0, s1, s2, s3), isem, s, BPT_FULL)

    @pl.when(c == 0)
    def _():
        _half_pass(vx0, outx10, *args)
        _half_pass(vx1, outx11, *args)
        _half_pass(outx10, outx20, *args)
        _half_pass(outx11, outx21, *args)

    @pl.when(c == 1)
    def _():
        _half_pass(vh0, outh10, *args)
        _half_pass(vh1, outh11, *args)
        _half_pass(outh10, outh20, *args)
        _half_pass(outh11, outh21, *args)


@functools.cache
def _s2_call():
    return pl.kernel(
        _s2_body, mesh=_mesh(), compiler_params=_sc_params(),
        out_type=[jax.ShapeDtypeStruct((NPAD, DH), _f32)] * 8,
        scratch_types=[
            pltpu.VMEM((2, CHK, BLK), _i32),
            pltpu.VMEM((2, CHK, BLK), _i32),
            pltpu.VMEM((2, CHK, BLK), _f32),
            pltpu.VMEM((BLK, DH), _f32),
            pltpu.VMEM((BLK, DH), _f32),
            pltpu.VMEM((BLK, DH), _f32),
            pltpu.VMEM((BLK, DH), _f32),
            pltpu.VMEM_SHARED((NPAD, DH), _f32),
            pltpu.VMEM_SHARED((NPAD, DH), _f32),
            pltpu.SemaphoreType.DMA,
            pltpu.SemaphoreType.DMA,
            pltpu.SemaphoreType.DMA,
            pltpu.SemaphoreType.DMA,
            pltpu.SemaphoreType.DMA,
            pltpu.SemaphoreType.DMA,
            pltpu.SemaphoreType.DMA,
            pltpu.SemaphoreType.DMA,
            pltpu.SemaphoreType.DMA,
        ],
    )


def _s1_body(v0, v1, rowb, colb, normb, outp0, outp1,
             rowv2, colv2, normv2, rb0, rb1, rb2, rb3, tb, acc,
             g0, g1, g2, g3, s0, s1, s2, s3, isem):
    c = lax.axis_index("c")
    s = lax.axis_index("s")

    args = (rowb, colb, normb, (c * 16 + s) * BPT_HALF, rowv2, colv2, normv2,
            (rb0, rb1, rb2, rb3), tb, acc,
            (g0, g1, g2, g3), (s0, s1, s2, s3), isem, s, BPT_HALF)
    _half_pass(v0, outp0.at[c], *args)
    _half_pass(v1, outp1.at[c], *args)


@functools.cache
def _s1_call():
    return pl.kernel(
        _s1_body, mesh=_mesh(), compiler_params=_sc_params(),
        out_type=[jax.ShapeDtypeStruct((2, NPAD, DH), _f32)] * 2,
        scratch_types=[
            pltpu.VMEM((2, CHK, BLK), _i32),
            pltpu.VMEM((2, CHK, BLK), _i32),
            pltpu.VMEM((2, CHK, BLK), _f32),
            pltpu.VMEM((BLK, DH), _f32),
            pltpu.VMEM((BLK, DH), _f32),
            pltpu.VMEM((BLK, DH), _f32),
            pltpu.VMEM((BLK, DH), _f32),
            pltpu.VMEM_SHARED((NPAD, DH), _f32),
            pltpu.VMEM_SHARED((NPAD, DH), _f32),
            pltpu.SemaphoreType.DMA,
            pltpu.SemaphoreType.DMA,
            pltpu.SemaphoreType.DMA,
            pltpu.SemaphoreType.DMA,
            pltpu.SemaphoreType.DMA,
            pltpu.SemaphoreType.DMA,
            pltpu.SemaphoreType.DMA,
            pltpu.SemaphoreType.DMA,
            pltpu.SemaphoreType.DMA,
        ],
    )


def _addp_body(p0_ref, p1_ref, o0_ref, o1_ref):
    o0_ref[...] = p0_ref[0] + p0_ref[1]
    o1_ref[...] = p1_ref[0] + p1_ref[1]


def _add_partials(p0, p1):
    gb = NPAD // 1280
    spec_in = pl.BlockSpec((2, 1280, DH), lambda i: (0, i, 0))
    spec_out = pl.BlockSpec((1280, DH), lambda i: (i, 0))
    return pl.pallas_call(
        _addp_body,
        grid=(gb,),
        in_specs=[spec_in, spec_in],
        out_specs=[spec_out, spec_out],
        out_shape=[jax.ShapeDtypeStruct((NPAD, DH), _f32)] * 2,
    )(p0, p1)


def _dot_halves(x0, x1, w_ref):
    return (jnp.dot(x0, w_ref[0:DH, :], preferred_element_type=_f32)
            + jnp.dot(x1, w_ref[DH:2 * DH, :], preferred_element_type=_f32))


def _dense1_body(x_ref, a10_ref, a11_ref, a20_ref, a21_ref,
                 h_ref, b10_ref, b11_ref, b20_ref, b21_ref,
                 wx0_ref, wx1_ref, wx2_ref, wh0_ref, wh1_ref, wh2_ref,
                 bx_ref, bh_ref,
                 pi_ref, pf_ref, t00_ref, t01_ref):
    gx = jnp.dot(x_ref[...], wx0_ref[...], preferred_element_type=_f32)
    gx += _dot_halves(a10_ref[...], a11_ref[...], wx1_ref)
    gx += _dot_halves(a20_ref[...], a21_ref[...], wx2_ref)
    gx += bx_ref[...]
    gh = jnp.dot(h_ref[...], wh0_ref[...], preferred_element_type=_f32)
    gh += _dot_halves(b10_ref[...], b11_ref[...], wh1_ref)
    gh += _dot_halves(b20_ref[...], b21_ref[...], wh2_ref)
    gh += bh_ref[...]
    pi_ref[...] = gx[:, 0:D] + gh[:, 0:D]
    pf_ref[...] = gx[:, D:2 * D] + gh[:, D:2 * D]
    t00_ref[...] = gx[:, 2 * D:2 * D + DH]
    t01_ref[...] = gx[:, 2 * D + DH:3 * D]


def _dense1(x, a10, a11, a20, a21, h, b10, b11, b20, b21,
            wx0, wx1, wx2, wh0, wh1, wh2, bx, bh):
    grid = N // BN
    row_spec = pl.BlockSpec((BN, D), lambda i: (i, 0))
    half_spec = pl.BlockSpec((BN, DH), lambda i: (i, 0))
    full = lambda s: pl.BlockSpec(s, lambda i: tuple(0 for _ in s))
    return pl.pallas_call(
        _dense1_body,
        grid=(grid,),
        in_specs=[row_spec] + [half_spec] * 4 + [row_spec] + [half_spec] * 4
                 + [full((D, 3 * D))] * 3 + [full((D, 2 * D))] * 3
                 + [full((1, 3 * D)), full((1, 2 * D))],
        out_specs=[row_spec, row_spec, half_spec, half_spec],
        out_shape=[jax.ShapeDtypeStruct((N, D), _f32)] * 2
                  + [jax.ShapeDtypeStruct((N, DH), _f32)] * 2,
    )(x, a10, a11, a20, a21, h, b10, b11, b20, b21,
      wx0, wx1, wx2, wh0, wh1, wh2, bx, bh)


def _dense2_body(t00_ref, t01_ref, c10_ref, c11_ref, c2p0_ref, c2p1_ref,
                 pi_ref, pf_ref, c_ref, wci_ref, wcf_ref,
                 wt0_ref, wt1_ref, wt2_ref,
                 bt_ref, bi_ref, bf_ref, bc_ref, out_ref):
    t00 = t00_ref[...]
    t01 = t01_ref[...]
    th = _dot_halves(t00, t01, wt0_ref)
    th += _dot_halves(c10_ref[...], c11_ref[...], wt1_ref)
    th += _dot_halves(c2p0_ref[0] + c2p0_ref[1], c2p1_ref[0] + c2p1_ref[1],
                      wt2_ref)
    t0 = jnp.concatenate([t00, t01], axis=1)
    t = jnp.tanh(t0 + th + bt_ref[...] + bc_ref[...])
    c = c_ref[...]
    gi = jax.nn.sigmoid(pi_ref[...] + wci_ref[...] * c + bi_ref[...])
    gf = jax.nn.sigmoid(pf_ref[...] + wcf_ref[...] * c + bf_ref[...])
    out_ref[...] = gf * c + gi * t


def _dense2(t00, t01, c10, c11, c2p0, c2p1, p_i, p_f, c, w_ci, w_cf,
            wt0, wt1, wt2, bt, b_i, b_f, b_c):
    grid = N // BN
    row_spec = pl.BlockSpec((BN, D), lambda i: (i, 0))
    half_spec = pl.BlockSpec((BN, DH), lambda i: (i, 0))
    part_spec = pl.BlockSpec((2, BN, DH), lambda i: (0, i, 0))
    full = lambda s: pl.BlockSpec(s, lambda i: tuple(0 for _ in s))
    return pl.pallas_call(
        _dense2_body,
        grid=(grid,),
        in_specs=[half_spec, half_spec, half_spec, half_spec,
                  part_spec, part_spec,
                  row_spec, row_spec, row_spec, row_spec, row_spec]
                 + [full((D, D))] * 3 + [full((1, D))] * 4,
        out_specs=row_spec,
        out_shape=jax.ShapeDtypeStruct((N, D), _f32),
    )(t00, t01, c10, c11, c2p0, c2p1, p_i, p_f, c, w_ci, w_cf,
      wt0, wt1, wt2, bt, b_i, b_f, b_c)


def kernel(X, edge_index, edge_weight, H, C,
           conv_x_i_W, conv_x_i_b, conv_h_i_W, conv_h_i_b,
           conv_x_f_W, conv_x_f_b, conv_h_f_W, conv_h_f_b,
           conv_x_c_W, conv_x_c_b,
           w_ci, w_cf, b_i, b_f, b_c):
    wx = [jnp.concatenate([conv_x_i_W[k], conv_x_f_W[k], conv_x_c_W[k]], axis=1)
          for k in range(3)]
    wh = [jnp.concatenate([conv_h_i_W[k], conv_h_f_W[k]], axis=1) for k in range(3)]
    wx0, wx1, wx2 = wx[0] - wx[2], -wx[1], 2.0 * wx[2]
    wh0, wh1, wh2 = wh[0] - wh[2], -wh[1], 2.0 * wh[2]
    bx = jnp.concatenate([conv_x_i_b, conv_x_f_b, conv_x_c_b])[None, :]
    bh = jnp.concatenate([conv_h_i_b, conv_h_f_b])[None, :]
    wt0 = conv_h_f_W[0] - conv_h_f_W[2]
    wt1 = -conv_h_f_W[1]
    wt2 = 2.0 * conv_h_f_W[2]
    bt = conv_h_f_b[None, :]

    pad = EPAD - E
    rowb = jnp.pad(edge_index[0], (0, pad)).reshape(NBLK, BLK).astype(_i32)
    colb = jnp.pad(edge_index[1], (0, pad)).reshape(NBLK, BLK).astype(_i32)
    wb = jnp.pad(edge_weight, (0, pad)).reshape(NBLK, BLK)

    deg_p = _deg_call()(rowb, wb)
    normb = _norm_call()(rowb, colb, wb,
                         deg_p.reshape(2, NPAD // BLK, BLK))

    x0, x1 = X[:, :DH], X[:, DH:]
    h0, h1 = H[:, :DH], H[:, DH:]
    (a10, a11, a20, a21,
     b10, b11, b20, b21) = _s2_call()(x0, x1, h0, h1, rowb, colb, normb)

    p_i, p_f, t00, t01 = _dense1(X, a10, a11, a20, a21,
                                 H, b10, b11, b20, b21,
                                 wx0, wx1, wx2, wh0, wh1, wh2, bx, bh)

    c1p0, c1p1 = _s1_call()(t00, t01, rowb, colb, normb)
    c10, c11 = _add_partials(c1p0, c1p1)
    c2p0, c2p1 = _s1_call()(c10, c11, rowb, colb, normb)

    c_new = _dense2(t00, t01, c10, c11, c2p0, c2p1, p_i, p_f, C,
                    w_ci, w_cf, wt0, wt1, wt2, bt, b_i, b_f, b_c)
    return (H, c_new)


# speedup vs baseline: 1.0064x; 1.0064x over previous
import functools

import jax
import jax.numpy as jnp
from jax import lax
from jax.experimental import pallas as pl
from jax.experimental.pallas import tpu as pltpu
from jax.experimental.pallas import tpu_sc as plsc

N = 10000
E = 320000
D = 128
DH = 64
NPAD = 10240
BLK = 128
NBLK = 2560
EPAD = NBLK * BLK
BPT_FULL = NBLK // 16
BPT_HALF = NBLK // 32
ROWS_PT = NPAD // 16

BN = 1000

_f32 = jnp.float32
_i32 = jnp.int32


@functools.cache
def _mesh():
    return plsc.VectorSubcoreMesh(core_axis_name="c", subcore_axis_name="s")


@functools.cache
def _sc_params():
    return pltpu.CompilerParams(needs_layout_passes=False, use_tc_tiling_on_sc=False)


def _zero_vmem2d(buf, nrows, ncols):
    z = jnp.zeros((16,), _f32)

    def zr(i, _):
        for f in range(ncols // 16):
            buf[i, pl.ds(f * 16, 16)] = z
        return 0

    lax.fori_loop(0, nrows, zr, 0)


CHK = 16


def _scale_block(buf, normv2, cb, jl):
    def grp(g, _):
        for u in range(8):
            e = g * 8 + u
            nrm = plsc.load_gather(
                normv2, [jnp.full((16,), cb, _i32), jnp.full((16,), jl, _i32),
                         jnp.full((16,), e, _i32)])
            for f in range(DH // 16):
                sl = pl.ds(f * 16, 16)
                buf[e, sl] = buf[e, sl] * nrm
        return 0

    lax.fori_loop(0, BLK // 8, grp, 0)


def _half_pass(v_hbm, out_hbm, rowb, colb, normb, base, rowv2, colv2,
               normv2, rbufs, tb, acc, gsems, ssems, isem, s, nblocks):
    nchunks = nblocks // CHK
    _zero_vmem2d(rbufs[3], BLK, DH)
    for k in range(ROWS_PT // BLK):
        pltpu.sync_copy(rbufs[3], acc.at[pl.ds(s * ROWS_PT + k * BLK, BLK)])
    @pl.when(s < 15)
    def _():
        pltpu.sync_copy(v_hbm.at[pl.ds(s * ROWS_PT, ROWS_PT)],
                        tb.at[pl.ds(s * ROWS_PT, ROWS_PT)])

    @pl.when(s == 15)
    def _():
        pltpu.sync_copy(v_hbm.at[pl.ds(15 * ROWS_PT, N - 15 * ROWS_PT)],
                        tb.at[pl.ds(15 * ROWS_PT, N - 15 * ROWS_PT)])

    plsc.subcore_barrier()

    pltpu.sync_copy(rowb.at[pl.ds(base, CHK)], rowv2.at[0])
    pltpu.sync_copy(colb.at[pl.ds(base, CHK)], colv2.at[0])
    pltpu.sync_copy(normb.at[pl.ds(base, CHK)], normv2.at[0])
    pltpu.async_copy(tb.at[colv2.at[0, 0]], rbufs[0], gsems[0])
    pltpu.async_copy(tb.at[colv2.at[0, 1]], rbufs[1], gsems[1])

    def quad(p, _):
        i1 = (p >> 2) + 1
        cb1 = i1 & 1
        off = base + i1 * CHK

        @pl.when(((p & 3) == 0) & (i1 < nchunks))
        def _():
            pltpu.async_copy(rowb.at[pl.ds(off, CHK)], rowv2.at[cb1], isem)
            pltpu.async_copy(colb.at[pl.ds(off, CHK)], colv2.at[cb1], isem)
            pltpu.async_copy(normb.at[pl.ds(off, CHK)], normv2.at[cb1], isem)

        @pl.when(((p & 3) == 3) & (i1 < nchunks))
        def _():
            pltpu.make_async_copy(rowb.at[pl.ds(off, CHK)], rowv2.at[cb1],
                                  isem).wait()
            pltpu.make_async_copy(colb.at[pl.ds(off, CHK)], colv2.at[cb1],
                                  isem).wait()
            pltpu.make_async_copy(normb.at[pl.ds(off, CHK)], normv2.at[cb1],
                                  isem).wait()

        for u in range(4):
            q = 4 * p + u
            cb = (q >> 4) & 1
            jl = q & (CHK - 1)
            w = (u + 2) % 4
            pltpu.make_async_copy(tb.at[colv2.at[cb, jl]], rbufs[u],
                                  gsems[u]).wait()
            _scale_block(rbufs[u], normv2, cb, jl)
            pltpu.async_copy(rbufs[u], acc.at[rowv2.at[cb, jl]], ssems[u],
                             add=True)

            @pl.when(q >= 2)
            def _():
                pltpu.make_async_copy(rbufs[w], acc.at[pl.ds(0, BLK)],
                                      ssems[w]).wait()

            @pl.when(q + 2 < nblocks)
            def _():
                q2 = q + 2
                pltpu.async_copy(tb.at[colv2.at[(q2 >> 4) & 1,
                                                q2 & (CHK - 1)]],
                                 rbufs[w], gsems[w])
        return 0

    lax.fori_loop(0, nblocks // 4, quad, 0)
    pltpu.make_async_copy(rbufs[(nblocks - 2) % 4], acc.at[pl.ds(0, BLK)],
                          ssems[(nblocks - 2) % 4]).wait()
    pltpu.make_async_copy(rbufs[(nblocks - 1) % 4], acc.at[pl.ds(0, BLK)],
                          ssems[(nblocks - 1) % 4]).wait()
    plsc.subcore_barrier()
    pltpu.sync_copy(acc.at[pl.ds(s * ROWS_PT, ROWS_PT)],
                    out_hbm.at[pl.ds(s * ROWS_PT, ROWS_PT)])
    plsc.subcore_barrier()


def _deg_body(rowb, wb, out, rowv, wv, zbuf, acc, sem):
    c = lax.axis_index("c")
    s = lax.axis_index("s")
    z = jnp.zeros((16,), _f32)

    def zr(i, _):
        zbuf[pl.ds(i * 16, 16)] = z
        return 0

    lax.fori_loop(0, ROWS_PT // 16, zr, 0)
    pltpu.sync_copy(zbuf, acc.at[pl.ds(s * ROWS_PT, ROWS_PT)])
    plsc.subcore_barrier()
    base = (c * 16 + s) * BPT_HALF
    pltpu.sync_copy(rowb.at[pl.ds(base, BPT_HALF)], rowv)
    pltpu.sync_copy(wb.at[pl.ds(base, BPT_HALF)], wv)

    def block(j, _):
        pltpu.sync_copy(wv.at[j], acc.at[rowv.at[j]], add=True)
        return 0

    lax.fori_loop(0, BPT_HALF, block, 0)
    plsc.subcore_barrier()
    pltpu.sync_copy(acc.at[pl.ds(s * ROWS_PT, ROWS_PT)],
                    out.at[c, pl.ds(s * ROWS_PT, ROWS_PT)])


@functools.cache
def _deg_call():
    return pl.kernel(
        _deg_body, mesh=_mesh(), compiler_params=_sc_params(),
        out_type=jax.ShapeDtypeStruct((2, NPAD), _f32),
        scratch_types=[
            pltpu.VMEM((BPT_HALF, BLK), _i32),
            pltpu.VMEM((BPT_HALF, BLK), _f32),
            pltpu.VMEM((ROWS_PT,), _f32),
            pltpu.VMEM_SHARED((NPAD,), _f32),
            pltpu.SemaphoreType.DMA,
        ],
    )


def _norm_body(rowb, colb, wb, degp, out, rowv, colv, wv, dgv, dv, nb, sem):
    c = lax.axis_index("c")
    s = lax.axis_index("s")
    base = (c * 16 + s) * BPT_HALF
    pltpu.sync_copy(rowb.at[pl.ds(base, BPT_HALF)], rowv)
    pltpu.sync_copy(colb.at[pl.ds(base, BPT_HALF)], colv)
    pltpu.sync_copy(wb.at[pl.ds(base, BPT_HALF)], wv)
    pltpu.sync_copy(degp.at[0], dgv)
    pltpu.sync_copy(degp.at[1], dv)

    def dinv_grp(i, _):
        r = i >> 3
        g = i & 7
        sl = pl.ds(g * 16, 16)
        d = dgv[r, sl] + dv[r, sl]
        y = plsc.bitcast(
            jnp.int32(0x5F3759DF) - (plsc.bitcast(d, _i32) >> 1), _f32)
        h = -0.5 * d
        y = y * (1.5 + h * y * y)
        y = y * (1.5 + h * y * y)
        y = y * (1.5 + h * y * y)
        dv[r, sl] = jnp.where(d > 0, y, 0.0)
        return 0

    lax.fori_loop(0, (NPAD // BLK) * 8, dinv_grp, 0)

    def block(j, _):
        for g in range(8):
            sl = pl.ds(g * 16, 16)
            r = rowv[j, sl]
            cc = colv[j, sl]
            dr = plsc.load_gather(dv, [r >> 7, r & 127])
            dc = plsc.load_gather(dv, [cc >> 7, cc & 127])
            nb[j, sl] = dr * wv[j, sl] * dc
        return 0

    lax.fori_loop(0, BPT_HALF, block, 0)
    pltpu.sync_copy(nb, out.at[pl.ds(base, BPT_HALF)])


@functools.cache
def _norm_call():
    return pl.kernel(
        _norm_body, mesh=_mesh(), compiler_params=_sc_params(),
        out_type=jax.ShapeDtypeStruct((NBLK, BLK), _f32),
        scratch_types=[
            pltpu.VMEM((BPT_HALF, BLK), _i32),
            pltpu.VMEM((BPT_HALF, BLK), _i32),
            pltpu.VMEM((BPT_HALF, BLK), _f32),
            pltpu.VMEM((NPAD // BLK, BLK), _f32),
            pltpu.VMEM((NPAD // BLK, BLK), _f32),
            pltpu.VMEM((BPT_HALF, BLK), _f32),
            pltpu.SemaphoreType.DMA,
        ],
    )


def _s2_body(vx0, vx1, vh0, vh1, rowb, colb, normb,
             outx0, outx1, outh0, outh1,
             rowv2, colv2, normv2, rb0, rb1, rb2, rb3, tb, acc,
             g0, g1, g2, g3, s0, s1, s2, s3, isem):
    c = lax.axis_index("c")
    s = lax.axis_index("s")

    args = (rowb, colb, normb, s * BPT_FULL, rowv2, colv2, normv2,
            (rb0, rb1, rb2, rb3), tb, acc,
            (g0, g1, g2, g3), (s0, s1, s2, s3), isem, s, BPT_FULL)

    @pl.when(c == 0)
    def _():
        _half_pass(vx0, outx0, *args)
        _half_pass(vx1, outx1, *args)

    @pl.when(c == 1)
    def _():
        _half_pass(vh0, outh0, *args)
        _half_pass(vh1, outh1, *args)


@functools.cache
def _s2_call():
    return pl.kernel(
        _s2_body, mesh=_mesh(), compiler_params=_sc_params(),
        out_type=[jax.ShapeDtypeStruct((NPAD, DH), _f32)] * 4,
        scratch_types=[
            pltpu.VMEM((2, CHK, BLK), _i32),
            pltpu.VMEM((2, CHK, BLK), _i32),
            pltpu.VMEM((2, CHK, BLK), _f32),
            pltpu.VMEM((BLK, DH), _f32),
            pltpu.VMEM((BLK, DH), _f32),
            pltpu.VMEM((BLK, DH), _f32),
            pltpu.VMEM((BLK, DH), _f32),
            pltpu.VMEM_SHARED((NPAD, DH), _f32),
            pltpu.VMEM_SHARED((NPAD, DH), _f32),
            pltpu.SemaphoreType.DMA,
            pltpu.SemaphoreType.DMA,
            pltpu.SemaphoreType.DMA,
            pltpu.SemaphoreType.DMA,
            pltpu.SemaphoreType.DMA,
            pltpu.SemaphoreType.DMA,
            pltpu.SemaphoreType.DMA,
            pltpu.SemaphoreType.DMA,
            pltpu.SemaphoreType.DMA,
        ],
    )


def _s1_body(v0, v1, rowb, colb, normb, outp0, outp1,
             rowv2, colv2, normv2, rb0, rb1, rb2, rb3, tb, acc,
             g0, g1, g2, g3, s0, s1, s2, s3, isem):
    c = lax.axis_index("c")
    s = lax.axis_index("s")

    args = (rowb, colb, normb, (c * 16 + s) * BPT_HALF, rowv2, colv2, normv2,
            (rb0, rb1, rb2, rb3), tb, acc,
            (g0, g1, g2, g3), (s0, s1, s2, s3), isem, s, BPT_HALF)
    _half_pass(v0, outp0.at[c], *args)
    _half_pass(v1, outp1.at[c], *args)


@functools.cache
def _s1_call():
    return pl.kernel(
        _s1_body, mesh=_mesh(), compiler_params=_sc_params(),
        out_type=[jax.ShapeDtypeStruct((2, NPAD, DH), _f32)] * 2,
        scratch_types=[
            pltpu.VMEM((2, CHK, BLK), _i32),
            pltpu.VMEM((2, CHK, BLK), _i32),
            pltpu.VMEM((2, CHK, BLK), _f32),
            pltpu.VMEM((BLK, DH), _f32),
            pltpu.VMEM((BLK, DH), _f32),
            pltpu.VMEM((BLK, DH), _f32),
            pltpu.VMEM((BLK, DH), _f32),
            pltpu.VMEM_SHARED((NPAD, DH), _f32),
            pltpu.VMEM_SHARED((NPAD, DH), _f32),
            pltpu.SemaphoreType.DMA,
            pltpu.SemaphoreType.DMA,
            pltpu.SemaphoreType.DMA,
            pltpu.SemaphoreType.DMA,
            pltpu.SemaphoreType.DMA,
            pltpu.SemaphoreType.DMA,
            pltpu.SemaphoreType.DMA,
            pltpu.SemaphoreType.DMA,
            pltpu.SemaphoreType.DMA,
        ],
    )


def _addp_body(p0_ref, p1_ref, o0_ref, o1_ref):
    o0_ref[...] = p0_ref[0] + p0_ref[1]
    o1_ref[...] = p1_ref[0] + p1_ref[1]


def _add_partials(p0, p1):
    gb = NPAD // 1280
    spec_in = pl.BlockSpec((2, 1280, DH), lambda i: (0, i, 0))
    spec_out = pl.BlockSpec((1280, DH), lambda i: (i, 0))
    return pl.pallas_call(
        _addp_body,
        grid=(gb,),
        in_specs=[spec_in, spec_in],
        out_specs=[spec_out, spec_out],
        out_shape=[jax.ShapeDtypeStruct((NPAD, DH), _f32)] * 2,
    )(p0, p1)


def _dot_halves(x0, x1, w_ref):
    return (jnp.dot(x0, w_ref[0:DH, :], preferred_element_type=_f32)
            + jnp.dot(x1, w_ref[DH:2 * DH, :], preferred_element_type=_f32))


def _dense1_body(x_ref, a10_ref, a11_ref, a20_ref, a21_ref,
                 h_ref, b10_ref, b11_ref, b20_ref, b21_ref,
                 wx0_ref, wx1_ref, wx2_ref, wh0_ref, wh1_ref, wh2_ref,
                 bx_ref, bh_ref,
                 pi_ref, pf_ref, t00_ref, t01_ref):
    gx = jnp.dot(x_ref[...], wx0_ref[...], preferred_element_type=_f32)
    gx += _dot_halves(a10_ref[...], a11_ref[...], wx1_ref)
    gx += _dot_halves(a20_ref[...], a21_ref[...], wx2_ref)
    gx += bx_ref[...]
    gh = jnp.dot(h_ref[...], wh0_ref[...], preferred_element_type=_f32)
    gh += _dot_halves(b10_ref[...], b11_ref[...], wh1_ref)
    gh += _dot_halves(b20_ref[...], b21_ref[...], wh2_ref)
    gh += bh_ref[...]
    pi_ref[...] = gx[:, 0:D] + gh[:, 0:D]
    pf_ref[...] = gx[:, D:2 * D] + gh[:, D:2 * D]
    t00_ref[...] = gx[:, 2 * D:2 * D + DH]
    t01_ref[...] = gx[:, 2 * D + DH:3 * D]


def _dense1(x, a10, a11, a20, a21, h, b10, b11, b20, b21,
            wx0, wx1, wx2, wh0, wh1, wh2, bx, bh):
    grid = N // BN
    row_spec = pl.BlockSpec((BN, D), lambda i: (i, 0))
    half_spec = pl.BlockSpec((BN, DH), lambda i: (i, 0))
    full = lambda s: pl.BlockSpec(s, lambda i: tuple(0 for _ in s))
    return pl.pallas_call(
        _dense1_body,
        grid=(grid,),
        in_specs=[row_spec] + [half_spec] * 4 + [row_spec] + [half_spec] * 4
                 + [full((D, 3 * D))] * 3 + [full((D, 2 * D))] * 3
                 + [full((1, 3 * D)), full((1, 2 * D))],
        out_specs=[row_spec, row_spec, half_spec, half_spec],
        out_shape=[jax.ShapeDtypeStruct((N, D), _f32)] * 2
                  + [jax.ShapeDtypeStruct((N, DH), _f32)] * 2,
    )(x, a10, a11, a20, a21, h, b10, b11, b20, b21,
      wx0, wx1, wx2, wh0, wh1, wh2, bx, bh)


def _dense2_body(t00_ref, t01_ref, c10_ref, c11_ref, c2p0_ref, c2p1_ref,
                 pi_ref, pf_ref, c_ref, wci_ref, wcf_ref,
                 wt0_ref, wt1_ref, wt2_ref,
                 bt_ref, bi_ref, bf_ref, bc_ref, out_ref):
    t00 = t00_ref[...]
    t01 = t01_ref[...]
    th = _dot_halves(t00, t01, wt0_ref)
    th += _dot_halves(c10_ref[...], c11_ref[...], wt1_ref)
    th += _dot_halves(c2p0_ref[0] + c2p0_ref[1], c2p1_ref[0] + c2p1_ref[1],
                      wt2_ref)
    t0 = jnp.concatenate([t00, t01], axis=1)
    t = jnp.tanh(t0 + th + bt_ref[...] + bc_ref[...])
    c = c_ref[...]
    gi = jax.nn.sigmoid(pi_ref[...] + wci_ref[...] * c + bi_ref[...])
    gf = jax.nn.sigmoid(pf_ref[...] + wcf_ref[...] * c + bf_ref[...])
    out_ref[...] = gf * c + gi * t


def _dense2(t00, t01, c10, c11, c2p0, c2p1, p_i, p_f, c, w_ci, w_cf,
            wt0, wt1, wt2, bt, b_i, b_f, b_c):
    grid = N // BN
    row_spec = pl.BlockSpec((BN, D), lambda i: (i, 0))
    half_spec = pl.BlockSpec((BN, DH), lambda i: (i, 0))
    part_spec = pl.BlockSpec((2, BN, DH), lambda i: (0, i, 0))
    full = lambda s: pl.BlockSpec(s, lambda i: tuple(0 for _ in s))
    return pl.pallas_call(
        _dense2_body,
        grid=(grid,),
        in_specs=[half_spec, half_spec, half_spec, half_spec,
                  part_spec, part_spec,
                  row_spec, row_spec, row_spec, row_spec, row_spec]
                 + [full((D, D))] * 3 + [full((1, D))] * 4,
        out_specs=row_spec,
        out_shape=jax.ShapeDtypeStruct((N, D), _f32),
    )(t00, t01, c10, c11, c2p0, c2p1, p_i, p_f, c, w_ci, w_cf,
      wt0, wt1, wt2, bt, b_i, b_f, b_c)


def kernel(X, edge_index, edge_weight, H, C,
           conv_x_i_W, conv_x_i_b, conv_h_i_W, conv_h_i_b,
           conv_x_f_W, conv_x_f_b, conv_h_f_W, conv_h_f_b,
           conv_x_c_W, conv_x_c_b,
           w_ci, w_cf, b_i, b_f, b_c):
    wx = [jnp.concatenate([conv_x_i_W[k], conv_x_f_W[k], conv_x_c_W[k]], axis=1)
          for k in range(3)]
    wh = [jnp.concatenate([conv_h_i_W[k], conv_h_f_W[k]], axis=1) for k in range(3)]
    wx0, wx1, wx2 = wx[0] - wx[2], -wx[1], 2.0 * wx[2]
    wh0, wh1, wh2 = wh[0] - wh[2], -wh[1], 2.0 * wh[2]
    bx = jnp.concatenate([conv_x_i_b, conv_x_f_b, conv_x_c_b])[None, :]
    bh = jnp.concatenate([conv_h_i_b, conv_h_f_b])[None, :]
    wt0 = conv_h_f_W[0] - conv_h_f_W[2]
    wt1 = -conv_h_f_W[1]
    wt2 = 2.0 * conv_h_f_W[2]
    bt = conv_h_f_b[None, :]

    pad = EPAD - E
    rowb = jnp.pad(edge_index[0], (0, pad)).reshape(NBLK, BLK).astype(_i32)
    colb = jnp.pad(edge_index[1], (0, pad)).reshape(NBLK, BLK).astype(_i32)
    wb = jnp.pad(edge_weight, (0, pad)).reshape(NBLK, BLK)

    deg_p = _deg_call()(rowb, wb)
    normb = _norm_call()(rowb, colb, wb,
                         deg_p.reshape(2, NPAD // BLK, BLK))

    x0, x1 = X[:, :DH], X[:, DH:]
    h0, h1 = H[:, :DH], H[:, DH:]
    a10, a11, b10, b11 = _s2_call()(x0, x1, h0, h1, rowb, colb, normb)
    a20, a21, b20, b21 = _s2_call()(a10, a11, b10, b11, rowb, colb, normb)

    p_i, p_f, t00, t01 = _dense1(X, a10, a11, a20, a21,
                                 H, b10, b11, b20, b21,
                                 wx0, wx1, wx2, wh0, wh1, wh2, bx, bh)

    c1p0, c1p1 = _s1_call()(t00, t01, rowb, colb, normb)
    c10, c11 = _add_partials(c1p0, c1p1)
    c2p0, c2p1 = _s1_call()(c10, c11, rowb, colb, normb)

    c_new = _dense2(t00, t01, c10, c11, c2p0, c2p1, p_i, p_f, C,
                    w_ci, w_cf, wt0, wt1, wt2, bt, b_i, b_f, b_c)
    return (H, c_new)
